# Initial kernel scaffold; baseline (speedup 1.0000x reference)
import jax
import jax.numpy as jnp
from jax.experimental import pallas as pl


def kernel(locs, init_embeddings, W, b):
    raise NotImplementedError("write your pallas kernel here")


# speedup vs baseline: 4.3876x; 4.3876x over previous
import jax
import jax.numpy as jnp
import numpy as np
from jax.experimental import pallas as pl

_B, _N, _K, _D = 8, 2048, 16, 128
_TILE = 256


def _knn_body(lxc, lyc, lxr, lyr, ew, bt, idx_out, attr_out):
    b = pl.program_id(0)
    i = pl.program_id(1)

    xc = lxc[0]
    yc = lyc[0]
    xr = lxr[0]
    yr = lyr[0]

    xcb = xc.astype(jnp.bfloat16).astype(jnp.float32)
    ycb = yc.astype(jnp.bfloat16).astype(jnp.float32)
    xrb = xr.astype(jnp.bfloat16).astype(jnp.float32)
    yrb = yr.astype(jnp.bfloat16).astype(jnp.float32)
    dot = xcb * xrb + ycb * yrb
    sqc = xc * xc + yc * yc
    sqr = xr * xr + yr * yr
    d2 = (sqc + sqr) - 2.0 * dot

    row_g = jax.lax.broadcasted_iota(jnp.int32, (_TILE, _N), 0) + i * _TILE
    col = jax.lax.broadcasted_iota(jnp.int32, (_TILE, _N), 1)
    vals = jnp.where(row_g == col, jnp.inf, jnp.maximum(d2, 1e-12))

    koi = jax.lax.broadcasted_iota(jnp.int32, (_TILE, _K), 1)
    dsel0 = jnp.zeros((_TILE, _K), jnp.float32)
    isel0 = jnp.zeros((_TILE, _K), jnp.int32)

    def body(j, carry):
        v, dsel, isel = carry
        m = jnp.min(v, axis=1, keepdims=True)
        t = jnp.where(v == m, col, _N)
        ij = jnp.min(t, axis=1, keepdims=True)
        v = jnp.where(t == ij, jnp.inf, v)
        oh = koi == j
        dsel = jnp.where(oh, m, dsel)
        isel = jnp.where(oh, ij, isel)
        return v, dsel, isel

    _, dsel, isel = jax.lax.fori_loop(0, _K, body, (vals, dsel0, isel0))

    idx_out[0] = isel + b * _N
    ed = jnp.sqrt(dsel)
    attr = jax.lax.dot_general(
        ed, ew[...],
        dimension_numbers=(((1,), (0,)), ((), ())),
        preferred_element_type=jnp.float32,
        precision=jax.lax.Precision.HIGHEST,
    )
    attr_out[0] = attr + bt[...]


def kernel(locs, init_embeddings, W, b):
    Bv, Nv, _ = locs.shape
    lxc = locs[:, :, 0:1]
    lyc = locs[:, :, 1:2]
    lxr = locs[:, :, 0].reshape(Bv, 1, Nv)
    lyr = locs[:, :, 1].reshape(Bv, 1, Nv)
    Wv = W.reshape(_D)
    ew = jnp.kron(jnp.eye(_K, dtype=jnp.float32), Wv[None, :])
    bt = jnp.tile(b, _K)[None, :]

    grid = (Bv, Nv // _TILE)
    idx_out, attr_out = pl.pallas_call(
        _knn_body,
        grid=grid,
        in_specs=[
            pl.BlockSpec((1, _TILE, 1), lambda b_, i: (b_, i, 0)),
            pl.BlockSpec((1, _TILE, 1), lambda b_, i: (b_, i, 0)),
            pl.BlockSpec((1, 1, Nv), lambda b_, i: (b_, 0, 0)),
            pl.BlockSpec((1, 1, Nv), lambda b_, i: (b_, 0, 0)),
            pl.BlockSpec((_K, _K * _D), lambda b_, i: (0, 0)),
            pl.BlockSpec((1, _K * _D), lambda b_, i: (0, 0)),
        ],
        out_specs=[
            pl.BlockSpec((1, _TILE, _K), lambda b_, i: (b_, i, 0)),
            pl.BlockSpec((1, _TILE, _K * _D), lambda b_, i: (b_, i, 0)),
        ],
        out_shape=[
            jax.ShapeDtypeStruct((Bv, Nv, _K), jnp.int32),
            jax.ShapeDtypeStruct((Bv, Nv, _K * _D), jnp.float32),
        ],
    )(lxc, lyc, lxr, lyr, ew, bt)

    x = init_embeddings.reshape(Bv * Nv, _D)
    src = jnp.broadcast_to(
        jnp.arange(Bv * Nv, dtype=jnp.int32)[:, None], (Bv * Nv, _K)
    ).reshape(-1)
    dst = idx_out.reshape(-1)
    edge_index = jnp.stack([src, dst], axis=0)
    edge_attr = attr_out.reshape(Bv * Nv * _K, _D)
    return x, edge_index, edge_attr


# speedup vs baseline: 9.4617x; 2.1564x over previous
import jax
import jax.numpy as jnp
import numpy as np
from jax.experimental import pallas as pl

_B, _N, _K, _D = 8, 2048, 16, 128
_TILE = 512


def _knn_body(lxc, lyc, lxr, lyr, ew, bt, idx_out, attr_out):
    b = pl.program_id(0)
    i = pl.program_id(1)

    xc = lxc[0]
    yc = lyc[0]
    xr = lxr[0]
    yr = lyr[0]

    xcb = xc.astype(jnp.bfloat16).astype(jnp.float32)
    ycb = yc.astype(jnp.bfloat16).astype(jnp.float32)
    xrb = xr.astype(jnp.bfloat16).astype(jnp.float32)
    yrb = yr.astype(jnp.bfloat16).astype(jnp.float32)
    dot = xcb * xrb + ycb * yrb
    sqc = xc * xc + yc * yc
    sqr = xr * xr + yr * yr
    d2 = (sqc + sqr) - 2.0 * dot

    row_g = jax.lax.broadcasted_iota(jnp.int32, (_TILE, _N), 0) + i * _TILE
    col = jax.lax.broadcasted_iota(jnp.int32, (_TILE, _N), 1)
    vals = jnp.where(row_g == col, jnp.inf, jnp.maximum(d2, 1e-12))

    key = (jax.lax.bitcast_convert_type(vals, jnp.int32)
           & jnp.int32(~0x7FF)) | col

    koi = jax.lax.broadcasted_iota(jnp.int32, (_TILE, _K), 1)
    ksel = jnp.zeros((_TILE, _K), jnp.int32)

    for j in range(_K):
        m = jnp.min(key, axis=1, keepdims=True)
        key = jnp.where(key == m, jnp.int32(0x7FFFFFFF), key)
        ksel = jnp.where(koi == j, m, ksel)

    isel = ksel & jnp.int32(0x7FF)
    dsel = jax.lax.bitcast_convert_type(ksel & jnp.int32(~0x7FF), jnp.float32)

    idx_out[0] = isel + b * _N
    ed = jnp.sqrt(dsel)
    attr = jax.lax.dot_general(
        ed, ew[...],
        dimension_numbers=(((1,), (0,)), ((), ())),
        preferred_element_type=jnp.float32,
        precision=jax.lax.Precision.HIGHEST,
    )
    attr_out[0] = attr + bt[...]


def kernel(locs, init_embeddings, W, b):
    Bv, Nv, _ = locs.shape
    lxc = locs[:, :, 0:1]
    lyc = locs[:, :, 1:2]
    lxr = locs[:, :, 0].reshape(Bv, 1, Nv)
    lyr = locs[:, :, 1].reshape(Bv, 1, Nv)
    Wv = W.reshape(_D)
    ew = jnp.kron(jnp.eye(_K, dtype=jnp.float32), Wv[None, :])
    bt = jnp.tile(b, _K)[None, :]

    grid = (Bv, Nv // _TILE)
    idx_out, attr_out = pl.pallas_call(
        _knn_body,
        grid=grid,
        in_specs=[
            pl.BlockSpec((1, _TILE, 1), lambda b_, i: (b_, i, 0)),
            pl.BlockSpec((1, _TILE, 1), lambda b_, i: (b_, i, 0)),
            pl.BlockSpec((1, 1, Nv), lambda b_, i: (b_, 0, 0)),
            pl.BlockSpec((1, 1, Nv), lambda b_, i: (b_, 0, 0)),
            pl.BlockSpec((_K, _K * _D), lambda b_, i: (0, 0)),
            pl.BlockSpec((1, _K * _D), lambda b_, i: (0, 0)),
        ],
        out_specs=[
            pl.BlockSpec((1, _TILE, _K), lambda b_, i: (b_, i, 0)),
            pl.BlockSpec((1, _TILE, _K * _D), lambda b_, i: (b_, i, 0)),
        ],
        out_shape=[
            jax.ShapeDtypeStruct((Bv, Nv, _K), jnp.int32),
            jax.ShapeDtypeStruct((Bv, Nv, _K * _D), jnp.float32),
        ],
    )(lxc, lyc, lxr, lyr, ew, bt)

    x = init_embeddings.reshape(Bv * Nv, _D)
    src = jnp.broadcast_to(
        jnp.arange(Bv * Nv, dtype=jnp.int32)[:, None], (Bv * Nv, _K)
    ).reshape(-1)
    dst = idx_out.reshape(-1)
    edge_index = jnp.stack([src, dst], axis=0)
    edge_attr = attr_out.reshape(Bv * Nv * _K, _D)
    return x, edge_index, edge_attr
